# baseline (speedup 1.0000x reference)
import functools

import jax
import jax.numpy as jnp
from jax import lax
from jax.experimental import pallas as pl
from jax.experimental.pallas import tpu as pltpu
from jax.experimental.pallas import tpu_sc as plsc

N_NODES = 10000
N_PAD = 10112
DUMMY = 10000
E_EDGES = 320000
E_PAD = 327680
CHUNK = 128
ROWS_PER_TILE = N_PAD // 16
DEG_CH = E_PAD // 2 // 16 // CHUNK
MAIN_CH = E_PAD // 2 // 16 // CHUNK

_SC_MESH = plsc.VectorSubcoreMesh(core_axis_name="c", subcore_axis_name="s")
_SC_PARAMS = pltpu.CompilerParams(use_tc_tiling_on_sc=False)


def _deg_body(dst_hbm, zeros16_hbm, ones_hbm, out_hbm, dst_v, ones_v, hist_sh):
    c = lax.axis_index("c")
    s = lax.axis_index("s")
    r0 = s * ROWS_PER_TILE
    pltpu.sync_copy(zeros16_hbm.at[pl.ds(r0, ROWS_PER_TILE)],
                    hist_sh.at[pl.ds(r0, ROWS_PER_TILE)])
    pltpu.sync_copy(dst_hbm.at[c, s], dst_v)
    pltpu.sync_copy(ones_hbm, ones_v)
    plsc.subcore_barrier()

    def step(k, carry):
        pltpu.sync_copy(ones_v, hist_sh.at[dst_v.at[k]], add=True)
        return carry

    lax.fori_loop(0, DEG_CH, step, 0)
    plsc.subcore_barrier()
    pltpu.sync_copy(hist_sh.at[pl.ds(r0, ROWS_PER_TILE)],
                    out_hbm.at[c].at[pl.ds(r0, ROWS_PER_TILE)])


_deg_kernel = functools.partial(
    pl.kernel,
    out_type=jax.ShapeDtypeStruct((2, N_PAD, 16), jnp.float32),
    mesh=_SC_MESH,
    compiler_params=_SC_PARAMS,
    scratch_types=[
        pltpu.VMEM((DEG_CH, CHUNK), jnp.int32),
        pltpu.VMEM((CHUNK, 16), jnp.float32),
        pltpu.VMEM_SHARED((N_PAD, 16), jnp.float32),
    ],
)(_deg_body)


NBUF = 8


def _scat_body(y_hbm, src_hbm, dst_hbm, zeros_hbm, out_hbm,
               src_v, dst_v, rows_v, gsems, ssems, z_sh):
    c = lax.axis_index("c")
    s = lax.axis_index("s")
    r0 = s * ROWS_PER_TILE
    pltpu.sync_copy(zeros_hbm.at[pl.ds(r0, ROWS_PER_TILE)],
                    z_sh.at[pl.ds(r0, ROWS_PER_TILE)])
    pltpu.sync_copy(src_hbm.at[c, s], src_v)
    pltpu.sync_copy(dst_hbm.at[c, s], dst_v)
    plsc.subcore_barrier()

    for b in range(NBUF - 1):
        pltpu.async_copy(y_hbm.at[src_v.at[b]], rows_v.at[b], gsems[b])

    def outer(j, carry):
        k0 = j * NBUF
        for b in range(NBUF):
            k = k0 + b
            pltpu.make_async_copy(y_hbm.at[src_v.at[k]],
                                  rows_v.at[b], gsems[b]).wait()
            pltpu.async_copy(rows_v.at[b], z_sh.at[dst_v.at[k]],
                             ssems[b], add=True)
            bp = (b + NBUF - 1) % NBUF
            nk = k + NBUF - 1

            @pl.when(jnp.logical_and(k >= 1, nk < MAIN_CH))
            def _():
                pltpu.make_async_copy(rows_v.at[bp],
                                      z_sh.at[dst_v.at[k - 1]],
                                      ssems[bp]).wait()
                pltpu.async_copy(y_hbm.at[src_v.at[nk]],
                                 rows_v.at[bp], gsems[bp])

            if b == 0:
                @pl.when(k == 0)
                def _():
                    pltpu.async_copy(y_hbm.at[src_v.at[NBUF - 1]],
                                     rows_v.at[NBUF - 1], gsems[NBUF - 1])
        return carry

    lax.fori_loop(0, MAIN_CH // NBUF, outer, 0)
    for b in range(NBUF):
        pltpu.make_async_copy(rows_v.at[b],
                              z_sh.at[dst_v.at[MAIN_CH - NBUF + b]],
                              ssems[b]).wait()
    plsc.subcore_barrier()
    pltpu.sync_copy(z_sh.at[pl.ds(r0, ROWS_PER_TILE)],
                    out_hbm.at[c].at[pl.ds(r0, ROWS_PER_TILE)])


_scat_kernel = functools.partial(
    pl.kernel,
    out_type=jax.ShapeDtypeStruct((2, N_PAD, 128), jnp.bfloat16),
    mesh=_SC_MESH,
    compiler_params=_SC_PARAMS,
    scratch_types=[
        pltpu.VMEM((MAIN_CH, CHUNK), jnp.int32),
        pltpu.VMEM((MAIN_CH, CHUNK), jnp.int32),
        pltpu.VMEM((NBUF, CHUNK, 128), jnp.bfloat16),
        [pltpu.SemaphoreType.DMA] * NBUF,
        [pltpu.SemaphoreType.DMA] * NBUF,
        pltpu.VMEM_SHARED((N_PAD, 128), jnp.bfloat16),
    ],
)(_scat_body)


def _dis_from_hist(hist_ref):
    deg = 1.0 + hist_ref[0, :, 0:1] + hist_ref[1, :, 0:1]
    return lax.rsqrt(deg)


def _t0_body(x_ref, w1_ref, hist_ref, y_ref, xw_ref):
    dis = _dis_from_hist(hist_ref)
    xw = jnp.dot(x_ref[...], w1_ref[...], preferred_element_type=jnp.float32)
    xw_ref[...] = xw
    y_ref[...] = (dis * xw).astype(jnp.bfloat16)


def _t0(x_pad, w1, hist):
    return pl.pallas_call(
        _t0_body,
        out_shape=(
            jax.ShapeDtypeStruct((N_PAD, 128), jnp.bfloat16),
            jax.ShapeDtypeStruct((N_PAD, 128), jnp.float32),
        ),
    )(x_pad, w1, hist)


def _mid_body(z_ref, xw_ref, hist_ref, w_ref, b_ref, y_ref, xw2_ref):
    dis = _dis_from_hist(hist_ref)
    zsum = (z_ref[0].astype(jnp.float32)
            + z_ref[1].astype(jnp.float32))
    pre = dis * zsum + (dis * dis) * xw_ref[...] + b_ref[...]
    h = jnp.maximum(pre, 0.0)
    row = lax.broadcasted_iota(jnp.int32, (N_PAD, 1), 0)
    h = jnp.where(row < N_NODES, h, 0.0)
    xw2 = jnp.dot(h, w_ref[...], preferred_element_type=jnp.float32)
    xw2_ref[...] = xw2
    y_ref[...] = (dis * xw2).astype(jnp.bfloat16)


def _t1(z, xw1, hist, w2, b1r):
    return pl.pallas_call(
        _mid_body,
        out_shape=(
            jax.ShapeDtypeStruct((N_PAD, 128), jnp.bfloat16),
            jax.ShapeDtypeStruct((N_PAD, 128), jnp.float32),
        ),
    )(z, xw1, hist, w2, b1r)


def _t2_body(z_ref, xw_ref, hist_ref, b_ref, wfc_ref, bfc_ref, out_ref):
    dis = _dis_from_hist(hist_ref)
    zsum = (z_ref[0].astype(jnp.float32)
            + z_ref[1].astype(jnp.float32))
    pre = dis * zsum + (dis * dis) * xw_ref[...] + b_ref[...]
    h = jnp.maximum(pre, 0.0)
    row = lax.broadcasted_iota(jnp.int32, (N_PAD, 1), 0)
    h = jnp.where(row < N_NODES, h, 0.0)
    g = jnp.sum(h, axis=0, keepdims=True) * (1.0 / N_NODES)
    logits = jnp.dot(g, wfc_ref[...],
                     preferred_element_type=jnp.float32) + bfc_ref[...]
    m = jnp.max(logits, axis=1, keepdims=True)
    lse = jnp.log(jnp.sum(jnp.exp(logits - m), axis=1, keepdims=True))
    out_ref[...] = logits - m - lse


def _t2(z, xw2, hist, b2r, wfc, bfcr):
    return pl.pallas_call(
        _t2_body,
        out_shape=jax.ShapeDtypeStruct((1, 5), jnp.float32),
    )(z, xw2, hist, b2r, wfc, bfcr)


def kernel(x, edge_index, W1, b1, W2, b2, Wfc, bfc):
    src = edge_index[0].astype(jnp.int32)
    dst = edge_index[1].astype(jnp.int32)
    pad = jnp.full((E_PAD - E_EDGES,), DUMMY, jnp.int32)
    srcp = jnp.concatenate([src, pad])
    dstp = jnp.concatenate([dst, pad])

    dst_deg = dstp.reshape(2, 16, DEG_CH, CHUNK)
    dst_es = dstp.reshape(2, 16, MAIN_CH, CHUNK)
    src_es = srcp.reshape(2, 16, MAIN_CH, CHUNK)

    zeros128 = jnp.zeros((N_PAD, 128), jnp.bfloat16)
    zeros16 = jnp.zeros((N_PAD, 16), jnp.float32)
    ones_rows = jnp.ones((CHUNK, 16), jnp.float32)
    x_pad = jnp.pad(x, ((0, N_PAD - N_NODES), (0, 0)))

    b1r = b1.reshape(1, 128)
    b2r = b2.reshape(1, 128)
    bfcr = bfc.reshape(1, 5)

    hist = _deg_kernel(dst_deg, zeros16, ones_rows)
    y1, xw1 = _t0(x_pad, W1, hist)
    z1 = _scat_kernel(y1, src_es, dst_es, zeros128)
    y2, xw2 = _t1(z1, xw1, hist, W2, b1r)
    z2 = _scat_kernel(y2, src_es, dst_es, zeros128)
    out = _t2(z2, xw2, hist, b2r, Wfc, bfcr)
    return out.reshape(5)


# speedup vs baseline: 1.5925x; 1.5925x over previous
import functools

import jax
import jax.numpy as jnp
from jax import lax
from jax.experimental import pallas as pl
from jax.experimental.pallas import tpu as pltpu
from jax.experimental.pallas import tpu_sc as plsc

N_NODES = 10000
N_PAD = 10112
DUMMY = 10000
E_EDGES = 320000
E_PAD = 327680
CHUNK = 128
ROWS_PER_TILE = N_PAD // 16
DEG_CH = E_PAD // 2 // 16 // CHUNK
MAIN_CH = E_PAD // 16 // CHUNK

_SC_MESH = plsc.VectorSubcoreMesh(core_axis_name="c", subcore_axis_name="s")
_SC_PARAMS = pltpu.CompilerParams(use_tc_tiling_on_sc=False)


def _deg_body(dst_hbm, zeros16_hbm, ones_hbm, out_hbm, dst_v, ones_v, hist_sh):
    c = lax.axis_index("c")
    s = lax.axis_index("s")
    r0 = s * ROWS_PER_TILE
    pltpu.sync_copy(zeros16_hbm.at[pl.ds(r0, ROWS_PER_TILE)],
                    hist_sh.at[pl.ds(r0, ROWS_PER_TILE)])
    pltpu.sync_copy(dst_hbm.at[c, s], dst_v)
    pltpu.sync_copy(ones_hbm, ones_v)
    plsc.subcore_barrier()

    def step(k, carry):
        pltpu.sync_copy(ones_v, hist_sh.at[dst_v.at[k]], add=True)
        return carry

    lax.fori_loop(0, DEG_CH, step, 0)
    plsc.subcore_barrier()
    pltpu.sync_copy(hist_sh.at[pl.ds(r0, ROWS_PER_TILE)],
                    out_hbm.at[c].at[pl.ds(r0, ROWS_PER_TILE)])


_deg_kernel = functools.partial(
    pl.kernel,
    out_type=jax.ShapeDtypeStruct((2, N_PAD, 16), jnp.float32),
    mesh=_SC_MESH,
    compiler_params=_SC_PARAMS,
    scratch_types=[
        pltpu.VMEM((DEG_CH, CHUNK), jnp.int32),
        pltpu.VMEM((CHUNK, 16), jnp.float32),
        pltpu.VMEM_SHARED((N_PAD, 16), jnp.float32),
    ],
)(_deg_body)


NBUF = 10


def _scat_body(y_hbm, src_hbm, dst_hbm, zeros_hbm, out_hbm,
               src_v, dst_v, rows_v, gsems, ssems, z_sh):
    c = lax.axis_index("c")
    s = lax.axis_index("s")
    r0 = s * ROWS_PER_TILE
    pltpu.sync_copy(zeros_hbm.at[pl.ds(r0, ROWS_PER_TILE)],
                    z_sh.at[pl.ds(r0, ROWS_PER_TILE)])
    pltpu.sync_copy(src_hbm.at[c, s], src_v)
    pltpu.sync_copy(dst_hbm.at[s], dst_v)
    plsc.subcore_barrier()

    for b in range(NBUF - 1):
        pltpu.async_copy(y_hbm.at[src_v.at[b]], rows_v.at[b], gsems[b])

    def outer(j, carry):
        k0 = j * NBUF
        for b in range(NBUF):
            k = k0 + b
            pltpu.make_async_copy(y_hbm.at[src_v.at[k]],
                                  rows_v.at[b], gsems[b]).wait()
            pltpu.async_copy(rows_v.at[b], z_sh.at[dst_v.at[k]],
                             ssems[b], add=True)
            bp = (b + NBUF - 1) % NBUF
            nk = k + NBUF - 1

            @pl.when(jnp.logical_and(k >= 1, nk < MAIN_CH))
            def _():
                pltpu.make_async_copy(rows_v.at[bp],
                                      z_sh.at[dst_v.at[k - 1]],
                                      ssems[bp]).wait()
                pltpu.async_copy(y_hbm.at[src_v.at[nk]],
                                 rows_v.at[bp], gsems[bp])

            if b == 0:
                @pl.when(k == 0)
                def _():
                    pltpu.async_copy(y_hbm.at[src_v.at[NBUF - 1]],
                                     rows_v.at[NBUF - 1], gsems[NBUF - 1])
        return carry

    lax.fori_loop(0, MAIN_CH // NBUF, outer, 0)
    for b in range(NBUF):
        pltpu.make_async_copy(rows_v.at[b],
                              z_sh.at[dst_v.at[MAIN_CH - NBUF + b]],
                              ssems[b]).wait()
    plsc.subcore_barrier()
    pltpu.sync_copy(z_sh.at[pl.ds(r0, ROWS_PER_TILE)],
                    out_hbm.at[c].at[pl.ds(r0, ROWS_PER_TILE)])


_scat_kernel = functools.partial(
    pl.kernel,
    out_type=jax.ShapeDtypeStruct((2, N_PAD, 64), jnp.bfloat16),
    mesh=_SC_MESH,
    compiler_params=_SC_PARAMS,
    scratch_types=[
        pltpu.VMEM((MAIN_CH, CHUNK), jnp.int32),
        pltpu.VMEM((MAIN_CH, CHUNK), jnp.int32),
        pltpu.VMEM((NBUF, CHUNK, 64), jnp.bfloat16),
        [pltpu.SemaphoreType.DMA] * NBUF,
        [pltpu.SemaphoreType.DMA] * NBUF,
        pltpu.VMEM_SHARED((N_PAD, 64), jnp.bfloat16),
    ],
)(_scat_body)


def _dis_from_hist(hist_ref):
    deg = 1.0 + hist_ref[0, :, 0:1] + hist_ref[1, :, 0:1]
    return lax.rsqrt(deg)


def _t0_body(x_ref, w1_ref, hist_ref, y_ref, xw_ref):
    dis = _dis_from_hist(hist_ref)
    xw = jnp.dot(x_ref[...], w1_ref[...], preferred_element_type=jnp.float32)
    xw_ref[...] = xw
    y = (dis * xw).astype(jnp.bfloat16)
    y_ref[0] = y[:, :64]
    y_ref[1] = y[:, 64:]


def _t0(x_pad, w1, hist):
    return pl.pallas_call(
        _t0_body,
        out_shape=(
            jax.ShapeDtypeStruct((2, N_PAD, 64), jnp.bfloat16),
            jax.ShapeDtypeStruct((N_PAD, 128), jnp.float32),
        ),
    )(x_pad, w1, hist)


def _mid_body(z_ref, xw_ref, hist_ref, w_ref, b_ref, y_ref, xw2_ref):
    dis = _dis_from_hist(hist_ref)
    zcat = jnp.concatenate([z_ref[0], z_ref[1]],
                           axis=1).astype(jnp.float32)
    pre = dis * zcat + (dis * dis) * xw_ref[...] + b_ref[...]
    h = jnp.maximum(pre, 0.0)
    row = lax.broadcasted_iota(jnp.int32, (N_PAD, 1), 0)
    h = jnp.where(row < N_NODES, h, 0.0)
    xw2 = jnp.dot(h, w_ref[...], preferred_element_type=jnp.float32)
    xw2_ref[...] = xw2
    y2 = (dis * xw2).astype(jnp.bfloat16)
    y_ref[0] = y2[:, :64]
    y_ref[1] = y2[:, 64:]


def _t1(z, xw1, hist, w2, b1r):
    return pl.pallas_call(
        _mid_body,
        out_shape=(
            jax.ShapeDtypeStruct((2, N_PAD, 64), jnp.bfloat16),
            jax.ShapeDtypeStruct((N_PAD, 128), jnp.float32),
        ),
    )(z, xw1, hist, w2, b1r)


def _t2_body(z_ref, xw_ref, hist_ref, b_ref, wfc_ref, bfc_ref, out_ref):
    dis = _dis_from_hist(hist_ref)
    zcat = jnp.concatenate([z_ref[0], z_ref[1]],
                           axis=1).astype(jnp.float32)
    pre = dis * zcat + (dis * dis) * xw_ref[...] + b_ref[...]
    h = jnp.maximum(pre, 0.0)
    row = lax.broadcasted_iota(jnp.int32, (N_PAD, 1), 0)
    h = jnp.where(row < N_NODES, h, 0.0)
    g = jnp.sum(h, axis=0, keepdims=True) * (1.0 / N_NODES)
    logits = jnp.dot(g, wfc_ref[...],
                     preferred_element_type=jnp.float32) + bfc_ref[...]
    m = jnp.max(logits, axis=1, keepdims=True)
    lse = jnp.log(jnp.sum(jnp.exp(logits - m), axis=1, keepdims=True))
    out_ref[...] = logits - m - lse


def _t2(z, xw2, hist, b2r, wfc, bfcr):
    return pl.pallas_call(
        _t2_body,
        out_shape=jax.ShapeDtypeStruct((1, 5), jnp.float32),
    )(z, xw2, hist, b2r, wfc, bfcr)


def kernel(x, edge_index, W1, b1, W2, b2, Wfc, bfc):
    src = edge_index[0].astype(jnp.int32)
    dst = edge_index[1].astype(jnp.int32)
    pad = jnp.full((E_PAD - E_EDGES,), DUMMY, jnp.int32)
    srcp = jnp.concatenate([src, pad])
    dstp = jnp.concatenate([dst, pad])

    dst_deg = dstp.reshape(2, 16, DEG_CH, CHUNK)
    dst_main = dstp.reshape(16, MAIN_CH, CHUNK)
    src_main = srcp.reshape(16, MAIN_CH, CHUNK)
    src2 = jnp.stack([src_main, src_main + N_PAD])

    zeros64 = jnp.zeros((N_PAD, 64), jnp.bfloat16)
    zeros16 = jnp.zeros((N_PAD, 16), jnp.float32)
    ones_rows = jnp.ones((CHUNK, 16), jnp.float32)
    x_pad = jnp.pad(x, ((0, N_PAD - N_NODES), (0, 0)))

    b1r = b1.reshape(1, 128)
    b2r = b2.reshape(1, 128)
    bfcr = bfc.reshape(1, 5)

    hist = _deg_kernel(dst_deg, zeros16, ones_rows)
    y1, xw1 = _t0(x_pad, W1, hist)
    z1 = _scat_kernel(y1.reshape(2 * N_PAD, 64), src2, dst_main, zeros64)
    y2, xw2 = _t1(z1, xw1, hist, W2, b1r)
    z2 = _scat_kernel(y2.reshape(2 * N_PAD, 64), src2, dst_main, zeros64)
    out = _t2(z2, xw2, hist, b2r, Wfc, bfcr)
    return out.reshape(5)
